# baseline (speedup 1.0000x reference)
import functools

import jax
import jax.numpy as jnp
from jax.experimental import pallas as pl
from jax.experimental.pallas import tpu as pltpu
from jax.experimental.pallas import tpu_sc as plsc

B = 4096
L = 200
D = 64

_BB = 64
_NW = 32
_CHUNK = 512


def _sc_gather_pairs(table_pairs, pair_idx):
    n = pair_idx.shape[0]
    b_per_w = n // _NW
    n_chunks = b_per_w // _CHUNK
    mesh = plsc.VectorSubcoreMesh(core_axis_name="c", subcore_axis_name="s")

    @functools.partial(
        pl.kernel,
        out_type=jax.ShapeDtypeStruct((n, 2 * D), table_pairs.dtype),
        mesh=mesh,
        scratch_types=[
            pltpu.VMEM((_CHUNK,), jnp.int32),
            pltpu.VMEM((_CHUNK, 2 * D), jnp.float32),
            pltpu.SemaphoreType.DMA,
        ],
    )
    def gather_kernel(tab_hbm, idx_hbm, out_hbm, idx_v, rows_v, sem):
        wid = jax.lax.axis_index("s") * 2 + jax.lax.axis_index("c")
        base = wid * b_per_w

        @pl.loop(0, n_chunks)
        def _(i):
            off = base + i * _CHUNK
            pltpu.sync_copy(idx_hbm.at[pl.ds(off, _CHUNK)], idx_v)
            pltpu.async_copy(tab_hbm.at[idx_v], rows_v, sem).wait()
            pltpu.sync_copy(rows_v, out_hbm.at[pl.ds(off, _CHUNK)])

    return gather_kernel(table_pairs, pair_idx)


def _att_body(hist_ref, seq_ref, cand_ref, candp_ref, prior_ref, label_ref,
              qw_ref, qb_ref, kwt_ref, kb_ref, vw_ref, vb_ref,
              pw_ref, pb_ref, cw_ref, cb_ref, out_ref):
    seq = seq_ref[...]
    h2 = hist_ref[...]
    emb = jnp.where((seq & 1)[:, :, None] == 1, h2[:, :, D:], h2[:, :, :D])
    c2 = cand_ref[...]
    ce = jnp.where(candp_ref[...] == 1, c2[:, D:], c2[:, :D])

    q = jnp.dot(ce, qw_ref[...], preferred_element_type=jnp.float32) + qb_ref[...]
    qp = jnp.dot(q, kwt_ref[...], preferred_element_type=jnp.float32)
    c = jnp.sum(q * kb_ref[...], axis=1, keepdims=True)

    logits_att = jnp.sum(emb * qp[:, None, :], axis=2) + c
    logits_att = jnp.where(seq != 0, logits_att, logits_att * (-(2.0 ** 32)))

    m = jnp.max(logits_att, axis=1, keepdims=True)
    e = jnp.exp(logits_att - m)
    s = e / jnp.sum(e, axis=1, keepdims=True)

    pooled = jnp.sum(s[:, :, None] * emb, axis=1)
    agg = jnp.dot(pooled, vw_ref[...], preferred_element_type=jnp.float32) + vb_ref[...]
    h = jnp.dot(agg, pw_ref[...], preferred_element_type=jnp.float32) + pb_ref[...]
    lr = jnp.dot(h, cw_ref[...], preferred_element_type=jnp.float32) + cb_ref[...]

    sc = prior_ref[...]
    s0 = (1.0 - sc) * (1.0 - 0.001) + 0.0001
    s1 = sc * (1.0 - 0.001) + 0.0001
    l0 = lr[:, 0:1] + (-jnp.log(1.0 / s0 - 1.0))
    l1 = lr[:, 1:2] + (-jnp.log(1.0 / s1 - 1.0))
    mm = jnp.maximum(l0, l1)
    lse = mm + jnp.log(jnp.exp(l0 - mm) + jnp.exp(l1 - mm))
    lab = label_ref[...].astype(jnp.float32)
    lp_sel = jnp.where(lab > 0.5, l1, l0) - lse
    out_ref[...] = jnp.broadcast_to(-jnp.sum(lp_sel), (1, 1, 1))


def _attention(hist2, hist_seq, cand2, cand_par, prior_score, label,
               Q_w, Q_b, K_wT, K_b, V_w, V_b, P_w, P_b, C_w, C_b):
    grid = B // _BB
    full = lambda shape: pl.BlockSpec(shape, lambda i: (0,) * len(shape))
    partials = pl.pallas_call(
        _att_body,
        grid=(grid,),
        in_specs=[
            pl.BlockSpec((_BB, L, 2 * D), lambda i: (i, 0, 0)),
            pl.BlockSpec((_BB, L), lambda i: (i, 0)),
            pl.BlockSpec((_BB, 2 * D), lambda i: (i, 0)),
            pl.BlockSpec((_BB, 1), lambda i: (i, 0)),
            pl.BlockSpec((_BB, 1), lambda i: (i, 0)),
            pl.BlockSpec((_BB, 1), lambda i: (i, 0)),
            full((D, D)), full((1, D)),
            full((D, D)), full((1, D)),
            full((D, D)), full((1, D)),
            full((D, D)), full((1, D)),
            full((D, 2)), full((1, 2)),
        ],
        out_specs=pl.BlockSpec((1, 1, 1), lambda i: (i, 0, 0)),
        out_shape=jax.ShapeDtypeStruct((grid, 1, 1), jnp.float32),
    )(hist2, hist_seq, cand2, cand_par, prior_score, label,
      Q_w, Q_b, K_wT, K_b, V_w, V_b, P_w, P_b, C_w, C_b)
    return jnp.sum(partials) / B


def kernel(hist_seq, cand, prior_score, label, emb_table,
           Q_w, Q_b, K_w, K_b, V_w, V_b, P_w, P_b, C_w, C_b):
    v = emb_table.shape[0]
    table_pairs = emb_table.reshape(v // 2, 2 * D)
    n_used = B * L + B
    n_pad = -n_used % (_NW * _CHUNK)
    pad_idx = (jnp.arange(n_pad, dtype=jnp.int32) % (v - 1)) + 1
    idx = jnp.concatenate([hist_seq.reshape(-1), cand, pad_idx]).astype(jnp.int32)
    gathered = _sc_gather_pairs(table_pairs, idx >> 1)
    hist2 = gathered[: B * L].reshape(B, L, 2 * D)
    cand2 = gathered[B * L: n_used]
    return _attention(
        hist2, hist_seq, cand2, (cand & 1).reshape(B, 1),
        prior_score.reshape(B, 1), label.reshape(B, 1).astype(jnp.int32),
        Q_w, Q_b.reshape(1, D), K_w.T, K_b.reshape(1, D),
        V_w, V_b.reshape(1, D), P_w, P_b.reshape(1, D),
        C_w, C_b.reshape(1, 2),
    )


# speedup vs baseline: 1.1948x; 1.1948x over previous
import functools

import jax
import jax.numpy as jnp
from jax.experimental import pallas as pl
from jax.experimental.pallas import tpu as pltpu
from jax.experimental.pallas import tpu_sc as plsc

B = 4096
L = 200
D = 64

_BB = 64
_NW = 32
_CHUNK = 256
_NCAND = 16384


def _sc_gather_pairs(table_pairs, hist_idx, cand_idx):
    n_hist = hist_idx.shape[0]
    hist_per_w = n_hist // _NW
    hist_chunks = hist_per_w // _CHUNK
    mesh = plsc.VectorSubcoreMesh(core_axis_name="c", subcore_axis_name="s")

    @functools.partial(
        pl.kernel,
        out_type=(
            jax.ShapeDtypeStruct((n_hist, 2 * D), table_pairs.dtype),
            jax.ShapeDtypeStruct((_NCAND, 2 * D), table_pairs.dtype),
        ),
        mesh=mesh,
        scratch_types=[
            pltpu.VMEM((_CHUNK,), jnp.int32),
            pltpu.VMEM((_CHUNK,), jnp.int32),
            pltpu.VMEM((_CHUNK, 2 * D), jnp.float32),
            pltpu.VMEM((_CHUNK, 2 * D), jnp.float32),
            pltpu.SemaphoreType.DMA,
            pltpu.SemaphoreType.DMA,
            pltpu.SemaphoreType.DMA,
            pltpu.SemaphoreType.DMA,
            pltpu.SemaphoreType.DMA,
            pltpu.SemaphoreType.DMA,
        ],
    )
    def gather_kernel(tab_hbm, hidx_hbm, cidx_hbm, oh_hbm, oc_hbm,
                      idx_v0, idx_v1, rows_v0, rows_v1,
                      si0, si1, sg0, sg1, sw0, sw1):
        wid = jax.lax.axis_index("s") * 2 + jax.lax.axis_index("c")

        def pair(idx_hbm, out_hbm, b0):
            b1 = b0 + _CHUNK
            c0 = pltpu.async_copy(idx_hbm.at[pl.ds(b0, _CHUNK)], idx_v0, si0)
            c1 = pltpu.async_copy(idx_hbm.at[pl.ds(b1, _CHUNK)], idx_v1, si1)
            c0.wait()
            g0 = pltpu.async_copy(tab_hbm.at[idx_v0], rows_v0, sg0)
            c1.wait()
            g1 = pltpu.async_copy(tab_hbm.at[idx_v1], rows_v1, sg1)
            g0.wait()
            w0 = pltpu.async_copy(rows_v0, out_hbm.at[pl.ds(b0, _CHUNK)], sw0)
            g1.wait()
            w1 = pltpu.async_copy(rows_v1, out_hbm.at[pl.ds(b1, _CHUNK)], sw1)
            w0.wait()
            w1.wait()

        hbase = wid * hist_per_w

        @pl.loop(0, hist_chunks // 2)
        def _(j):
            pair(hidx_hbm, oh_hbm, hbase + 2 * j * _CHUNK)

        pair(cidx_hbm, oc_hbm, wid * 2 * _CHUNK)

    return gather_kernel(table_pairs, hist_idx, cand_idx)


def _att_body(hist_ref, seq_ref, cand_ref, candp_ref, prior_ref, label_ref,
              qw_ref, qb_ref, kwt_ref, kb_ref, vw_ref, vb_ref,
              pw_ref, pb_ref, cw_ref, cb_ref, out_ref):
    seq3 = seq_ref[...]
    h2 = hist_ref[...]
    emb = jnp.where((seq3 & 1) == 1, h2[:, :, D:], h2[:, :, :D])

    c2 = cand_ref[...]
    ce = jnp.where(candp_ref[...] == 1, c2[:, D:], c2[:, :D])

    q = jnp.dot(ce, qw_ref[...], preferred_element_type=jnp.float32) + qb_ref[...]
    qp = jnp.dot(q, kwt_ref[...], preferred_element_type=jnp.float32)
    c = jnp.sum(q * kb_ref[...], axis=1, keepdims=True)

    prod = emb * qp[:, None, :]
    ones = jnp.full((D, D), 1.0, jnp.float32)
    la = jnp.dot(prod.reshape(_BB * L, D), ones,
                 preferred_element_type=jnp.float32).reshape(_BB, L, D)
    la = la + c[:, :, None]
    la = jnp.where(seq3 != 0, la, la * (-(2.0 ** 32)))

    m = jnp.max(la, axis=1, keepdims=True)
    e = jnp.exp(la - m)
    s = e * (1.0 / jnp.sum(e, axis=1, keepdims=True))

    pooled = jnp.sum(s * emb, axis=1)
    agg = jnp.dot(pooled, vw_ref[...], preferred_element_type=jnp.float32) + vb_ref[...]
    h = jnp.dot(agg, pw_ref[...], preferred_element_type=jnp.float32) + pb_ref[...]
    lr = jnp.dot(h, cw_ref[...], preferred_element_type=jnp.float32) + cb_ref[...]

    sc = prior_ref[...]
    s0 = (1.0 - sc) * (1.0 - 0.001) + 0.0001
    s1 = sc * (1.0 - 0.001) + 0.0001
    l0 = lr[:, 0:1] + (-jnp.log(1.0 / s0 - 1.0))
    l1 = lr[:, 1:2] + (-jnp.log(1.0 / s1 - 1.0))
    mm = jnp.maximum(l0, l1)
    lse = mm + jnp.log(jnp.exp(l0 - mm) + jnp.exp(l1 - mm))
    lab = label_ref[...].astype(jnp.float32)
    lp_sel = jnp.where(lab > 0.5, l1, l0) - lse
    out_ref[...] = jnp.broadcast_to(-jnp.sum(lp_sel), (1, 1, 1))


def _attention(hist2, seq3, cand2, cand_par, prior_score, label,
               Q_w, Q_b, K_wT, K_b, V_w, V_b, P_w, P_b, C_w, C_b):
    grid = B // _BB
    full = lambda shape: pl.BlockSpec(shape, lambda i: (0,) * len(shape))
    partials = pl.pallas_call(
        _att_body,
        grid=(grid,),
        in_specs=[
            pl.BlockSpec((_BB, L, 2 * D), lambda i: (i, 0, 0)),
            pl.BlockSpec((_BB, L, 1), lambda i: (i, 0, 0)),
            pl.BlockSpec((_BB, 2 * D), lambda i: (i, 0)),
            pl.BlockSpec((_BB, 1), lambda i: (i, 0)),
            pl.BlockSpec((_BB, 1), lambda i: (i, 0)),
            pl.BlockSpec((_BB, 1), lambda i: (i, 0)),
            full((D, D)), full((1, D)),
            full((D, D)), full((1, D)),
            full((D, D)), full((1, D)),
            full((D, D)), full((1, D)),
            full((D, 2)), full((1, 2)),
        ],
        out_specs=pl.BlockSpec((1, 1, 1), lambda i: (i, 0, 0)),
        out_shape=jax.ShapeDtypeStruct((grid, 1, 1), jnp.float32),
    )(hist2, seq3, cand2, cand_par, prior_score, label,
      Q_w, Q_b, K_wT, K_b, V_w, V_b, P_w, P_b, C_w, C_b)
    return jnp.sum(partials) / B


def kernel(hist_seq, cand, prior_score, label, emb_table,
           Q_w, Q_b, K_w, K_b, V_w, V_b, P_w, P_b, C_w, C_b):
    v = emb_table.shape[0]
    table_pairs = emb_table.reshape(v // 2, 2 * D)
    pad_idx = (jnp.arange(_NCAND - B, dtype=jnp.int32) % (v - 1)) + 1
    hist_idx = hist_seq.reshape(-1).astype(jnp.int32) >> 1
    cand_idx = jnp.concatenate([cand.astype(jnp.int32), pad_idx]) >> 1
    hist_rows, cand_rows = _sc_gather_pairs(table_pairs, hist_idx, cand_idx)
    hist2 = hist_rows.reshape(B, L, 2 * D)
    cand2 = cand_rows[:B]
    return _attention(
        hist2, hist_seq.reshape(B, L, 1), cand2, (cand & 1).reshape(B, 1),
        prior_score.reshape(B, 1), label.reshape(B, 1).astype(jnp.int32),
        Q_w, Q_b.reshape(1, D), K_w.T, K_b.reshape(1, D),
        V_w, V_b.reshape(1, D), P_w, P_b.reshape(1, D),
        C_w, C_b.reshape(1, 2),
    )
